# baseline (speedup 1.0000x reference)
import functools

import jax
import jax.numpy as jnp
from jax import lax
from jax.experimental import pallas as pl
from jax.experimental.pallas import tpu as pltpu
from jax.experimental.pallas import tpu_sc as plsc

N = 10000
E = 160000
D_IN = 256
D1 = 100
D1P = 112
D2 = 64
D3 = 32

NC = 2
NS = 16
NT = NC * NS
NPAD = 10240
SL = NPAD // NS
CHE = 128
NCHT = 40
EPT = NCHT * CHE
EPAD = NT * EPT

BR = 1024
GRID = NPAD // BR

_mesh = plsc.VectorSubcoreMesh(
    core_axis_name="c", subcore_axis_name="s", num_cores=NC, num_subcores=NS)
_sc_params = pltpu.CompilerParams(
    needs_layout_passes=False, use_tc_tiling_on_sc=False)


def _wid():
    return lax.axis_index("c") * NS + lax.axis_index("s")


def _zero_vmem_1d(ref, n):
    z = jnp.zeros((16,), jnp.float32)

    def body(i, _):
        ref[pl.ds(i * 16, 16)] = z
        return _

    lax.fori_loop(0, n // 16, body, None)


def _combine_and_store(hist, shared, red, outb, out_hbm):
    sid = lax.axis_index("s")
    cid = lax.axis_index("c")
    pltpu.sync_copy(hist, shared.at[sid])
    plsc.subcore_barrier()
    for k in range(NS):
        pltpu.sync_copy(shared.at[k, pl.ds(sid * SL, SL)], red.at[k])

    def body(j, _):
        sl = pl.ds(j * 16, 16)
        acc = red[0, sl]
        for k in range(1, NS):
            acc = acc + red[k, sl]
        outb[sl] = acc
        return _

    lax.fori_loop(0, SL // 16, body, None)
    pltpu.sync_copy(outb, out_hbm.at[cid, pl.ds(sid * SL, SL)])


@functools.partial(
    pl.kernel,
    out_type=jax.ShapeDtypeStruct((NC, NPAD), jnp.float32),
    mesh=_mesh,
    compiler_params=_sc_params,
    scratch_types=[
        pltpu.VMEM((NPAD,), jnp.float32),
        pltpu.VMEM((NCHT, CHE), jnp.int32),
        pltpu.VMEM_SHARED((NS, NPAD), jnp.float32),
        pltpu.VMEM((NS, SL), jnp.float32),
        pltpu.VMEM((SL,), jnp.float32),
    ],
)
def _deg_call(dstc_hbm, out_hbm, hist, dif, shared, red, outb):
    wid = _wid()
    pltpu.sync_copy(dstc_hbm.at[wid], dif)
    _zero_vmem_1d(hist, NPAD)
    ones = jnp.ones((16,), jnp.float32)

    def body(i, _):
        def inner(j, _2):
            idx = dif[i, pl.ds(j * 16, 16)]
            plsc.addupdate_scatter(hist, [idx], ones)
            return _2

        lax.fori_loop(0, CHE // 16, inner, None)
        return _

    lax.fori_loop(0, NCHT, body, None)
    _combine_and_store(hist, shared, red, outb, out_hbm)


@functools.partial(
    pl.kernel,
    out_type=jax.ShapeDtypeStruct((NC, NPAD), jnp.float32),
    mesh=_mesh,
    compiler_params=_sc_params,
    scratch_types=[
        pltpu.VMEM((NPAD,), jnp.float32),
        pltpu.VMEM((NPAD,), jnp.float32),
        pltpu.VMEM((NCHT, CHE), jnp.int32),
        pltpu.VMEM((NCHT, CHE), jnp.int32),
        pltpu.VMEM_SHARED((NS, NPAD), jnp.float32),
        pltpu.VMEM((NS, SL), jnp.float32),
        pltpu.VMEM((SL,), jnp.float32),
    ],
)
def _g_call(dinv_hbm, srcc_hbm, dstc_hbm, out_hbm, dtab, hist, sif, dif,
            shared, red, outb):
    wid = _wid()
    pltpu.sync_copy(srcc_hbm.at[wid], sif)
    pltpu.sync_copy(dstc_hbm.at[wid], dif)
    pltpu.sync_copy(dinv_hbm, dtab)
    _zero_vmem_1d(hist, NPAD)

    def body(i, _):
        def inner(j, _2):
            sl = pl.ds(j * 16, 16)
            vals = plsc.load_gather(dtab, [dif[i, sl]])
            plsc.addupdate_scatter(hist, [sif[i, sl]], vals)
            return _2

        lax.fori_loop(0, CHE // 16, inner, None)
        return _

    lax.fori_loop(0, NCHT, body, None)
    _combine_and_store(hist, shared, red, outb, out_hbm)

@functools.partial(
    pl.kernel,
    out_type=jax.ShapeDtypeStruct((NC, NPAD, D1P), jnp.float32),
    mesh=_mesh,
    compiler_params=_sc_params,
    scratch_types=[
        pltpu.VMEM((NCHT, CHE), jnp.int32),
        pltpu.VMEM((NCHT, CHE), jnp.int32),
        pltpu.VMEM((CHE, D1P), jnp.float32),
        pltpu.VMEM((CHE, D1P), jnp.float32),
        pltpu.SemaphoreType.DMA,
        pltpu.SemaphoreType.DMA,
        pltpu.VMEM_SHARED((NPAD, D1P), jnp.float32),
    ],
)
def _agg1_call(hp_hbm, srcc_hbm, dstc_hbm,
               zer_hbm, s_hbm, sidx, didx, rows0, rows1,
               sem0, sem1, acc):
    cid = lax.axis_index("c")
    sid = lax.axis_index("s")
    wid = cid * NS + sid
    pltpu.sync_copy(zer_hbm, acc.at[pl.ds(sid * SL, SL)])
    pltpu.sync_copy(srcc_hbm.at[wid], sidx)
    pltpu.sync_copy(dstc_hbm.at[wid], didx)
    plsc.subcore_barrier()

    pltpu.async_copy(hp_hbm.at[sidx.at[0]], rows0, sem0)

    def body(i, _):
        c0 = 2 * i
        c1 = c0 + 1
        pltpu.async_copy(hp_hbm.at[sidx.at[c1]], rows1, sem1)
        pltpu.make_async_copy(hp_hbm.at[sidx.at[c0]], rows0, sem0).wait()
        pltpu.sync_copy(rows0, acc.at[didx.at[c0]], add=True)

        @pl.when(i < NCHT // 2 - 1)
        def _nx():
            pltpu.async_copy(hp_hbm.at[sidx.at[c0 + 2]], rows0, sem0)

        pltpu.make_async_copy(hp_hbm.at[sidx.at[c1]], rows1, sem1).wait()
        pltpu.sync_copy(rows1, acc.at[didx.at[c1]], add=True)
        return _

    lax.fori_loop(0, NCHT // 2, body, None)
    plsc.subcore_barrier()
    pltpu.sync_copy(acc.at[pl.ds(sid * SL, SL)],
                    s_hbm.at[cid, pl.ds(sid * SL, SL)])


@functools.partial(
    pl.kernel,
    out_type=jax.ShapeDtypeStruct((NC, NPAD, D2), jnp.float32),
    mesh=_mesh,
    compiler_params=_sc_params,
    scratch_types=[
        pltpu.VMEM((NCHT, CHE), jnp.int32),
        pltpu.VMEM((NCHT, CHE), jnp.int32),
        pltpu.VMEM((CHE, D2), jnp.float32),
        pltpu.VMEM((CHE, D2), jnp.float32),
        pltpu.SemaphoreType.DMA,
        pltpu.SemaphoreType.DMA,
        pltpu.VMEM_SHARED((NPAD, D2), jnp.float32),
    ],
)
def _agg2_call(hp_hbm, srcc_hbm, dstc_hbm, zer_hbm, s_hbm, sidx, didx,
               rows0, rows1, sem0, sem1, acc):
    cid = lax.axis_index("c")
    sid = lax.axis_index("s")
    wid = cid * NS + sid
    pltpu.sync_copy(zer_hbm, acc.at[pl.ds(sid * SL, SL)])
    pltpu.sync_copy(srcc_hbm.at[wid], sidx)
    pltpu.sync_copy(dstc_hbm.at[wid], didx)
    plsc.subcore_barrier()

    pltpu.async_copy(hp_hbm.at[sidx.at[0]], rows0, sem0)

    def body(i, _):
        c0 = 2 * i
        c1 = c0 + 1
        pltpu.async_copy(hp_hbm.at[sidx.at[c1]], rows1, sem1)
        pltpu.make_async_copy(hp_hbm.at[sidx.at[c0]], rows0, sem0).wait()
        pltpu.sync_copy(rows0, acc.at[didx.at[c0]], add=True)

        @pl.when(i < NCHT // 2 - 1)
        def _nx():
            pltpu.async_copy(hp_hbm.at[sidx.at[c0 + 2]], rows0, sem0)

        pltpu.make_async_copy(hp_hbm.at[sidx.at[c1]], rows1, sem1).wait()
        pltpu.sync_copy(rows1, acc.at[didx.at[c1]], add=True)
        return _

    lax.fori_loop(0, NCHT // 2, body, None)
    plsc.subcore_barrier()
    pltpu.sync_copy(acc.at[pl.ds(sid * SL, SL)],
                    s_hbm.at[cid, pl.ds(sid * SL, SL)])


def _mm1_body(ca_ref, cb_ref, x_ref, w_ref, p_ref, dinv_ref):
    i = pl.program_id(0)
    row = lax.broadcasted_iota(jnp.int32, (BR, 1), 0) + i * BR
    deg = ca_ref[...] + cb_ref[...] + 1.0
    dv = jnp.where(row < N, lax.rsqrt(deg), 0.0)
    dinv_ref[...] = dv
    p_ref[...] = jnp.dot(x_ref[...], w_ref[...],
                         preferred_element_type=jnp.float32) * dv


def _mm2_body(sa_ref, sb_ref, p1_ref, dv_ref, b1_ref, w2_ref, out_ref):
    dv = dv_ref[...]
    h = dv * (sa_ref[...] + sb_ref[...] + p1_ref[...]) + b1_ref[...]
    h = jnp.maximum(h, 0.0)
    out_ref[...] = jnp.dot(h, w2_ref[...],
                           preferred_element_type=jnp.float32) * dv


def _fin_body(sa_ref, sb_ref, p2_ref, dv_ref, ga_ref, gb_ref, b2_ref, w3_ref,
              b3_ref, out_ref, acc_ref):
    i = pl.program_id(0)

    @pl.when(i == 0)
    def _z():
        acc_ref[...] = jnp.zeros_like(acc_ref)

    dv = dv_ref[...]
    h = jnp.maximum(dv * (sa_ref[...] + sb_ref[...] + p2_ref[...]) + b2_ref[...],
                    0.0)
    c = dv * (ga_ref[...] + gb_ref[...] + dv)
    acc_ref[...] += jnp.sum(c * h, axis=0, keepdims=True)

    @pl.when(i == GRID - 1)
    def _f():
        out_ref[...] = jnp.dot(acc_ref[...] * (1.0 / N), w3_ref[...],
                               preferred_element_type=jnp.float32) + b3_ref[...]


def _col_spec(d):
    return pl.BlockSpec((BR, d), lambda i: (i, 0))


def _const_spec(shape):
    return pl.BlockSpec(shape, lambda i: tuple(0 for _ in shape))


def kernel(x, edge_index, W1, b1, W2, b2, W3, b3):
    f32 = jnp.float32
    src = edge_index[0].astype(jnp.int32)
    dst = edge_index[1].astype(jnp.int32)
    npd = EPAD - E
    srcp = jnp.concatenate([src, jnp.zeros((npd,), jnp.int32)])
    dstp = jnp.concatenate([dst, jnp.full((npd,), NPAD - 1, jnp.int32)])
    srcc = srcp.reshape(NT, NCHT, CHE)
    dstc = dstp.reshape(NT, NCHT, CHE)
    xpad = jnp.zeros((NPAD, D_IN), f32).at[:N].set(x)
    W1p = jnp.zeros((D_IN, D1P), f32).at[:, :D1].set(W1)
    b1p = jnp.zeros((1, D1P), f32).at[0, :D1].set(b1)
    W2p = jnp.zeros((D1P, D2), f32).at[:D1].set(W2)

    cnt2 = _deg_call(dstc)

    P1p, dinv = pl.pallas_call(
        _mm1_body,
        grid=(GRID,),
        in_specs=[_col_spec(1), _col_spec(1), _col_spec(D_IN),
                  _const_spec((D_IN, D1P))],
        out_specs=[_col_spec(D1P), _col_spec(1)],
        out_shape=[jax.ShapeDtypeStruct((NPAD, D1P), f32),
                   jax.ShapeDtypeStruct((NPAD, 1), f32)],
    )(cnt2[0][:, None], cnt2[1][:, None], xpad, W1p)

    z1 = jnp.zeros((SL, D1P), f32)
    g2 = _g_call(dinv[:, 0], srcc, dstc)
    S1 = _agg1_call(P1p, srcc, dstc, z1)

    P2p = pl.pallas_call(
        _mm2_body,
        grid=(GRID,),
        in_specs=[_col_spec(D1P), _col_spec(D1P), _col_spec(D1P), _col_spec(1),
                  _const_spec((1, D1P)), _const_spec((D1P, D2))],
        out_specs=_col_spec(D2),
        out_shape=jax.ShapeDtypeStruct((NPAD, D2), f32),
    )(S1[0], S1[1], P1p, dinv, b1p, W2p)

    z2 = jnp.zeros((SL, D2), f32)
    S2 = _agg2_call(P2p, srcc, dstc, z2)

    out = pl.pallas_call(
        _fin_body,
        grid=(GRID,),
        in_specs=[_col_spec(D2), _col_spec(D2), _col_spec(D2), _col_spec(1),
                  _col_spec(1), _col_spec(1), _const_spec((1, D2)),
                  _const_spec((D2, D3)), _const_spec((1, D3))],
        out_specs=_const_spec((1, D3)),
        out_shape=jax.ShapeDtypeStruct((1, D3), f32),
        scratch_shapes=[pltpu.VMEM((1, D2), f32)],
    )(S2[0], S2[1], P2p, dinv, g2[0][:, None], g2[1][:, None],
      b2[None, :], W3, b3[None, :])

    return out


# speedup vs baseline: 1.9963x; 1.9963x over previous
import functools

import jax
import jax.numpy as jnp
from jax import lax
from jax.experimental import pallas as pl
from jax.experimental.pallas import tpu as pltpu
from jax.experimental.pallas import tpu_sc as plsc

N = 10000
E = 160000
D_IN = 256
D1 = 100
D1P = 112
D2 = 64
D3 = 32

NC = 2
NS = 16
NT = NC * NS
NPAD = 10240
SL = NPAD // NS
CHE = 128
NCHT = 40
EPT = NCHT * CHE
EPAD = NT * EPT

BR = 1024
GRID = NPAD // BR

_mesh = plsc.VectorSubcoreMesh(
    core_axis_name="c", subcore_axis_name="s", num_cores=NC, num_subcores=NS)
_sc_params = pltpu.CompilerParams(
    needs_layout_passes=False, use_tc_tiling_on_sc=False)


def _wid():
    return lax.axis_index("c") * NS + lax.axis_index("s")


def _zero_vmem_1d(ref, n):
    z = jnp.zeros((16,), jnp.float32)

    def body(i, _):
        ref[pl.ds(i * 16, 16)] = z
        return _

    lax.fori_loop(0, n // 16, body, None)


def _combine_and_store(hist, shared, red, outb, out_hbm):
    sid = lax.axis_index("s")
    cid = lax.axis_index("c")
    pltpu.sync_copy(hist, shared.at[sid])
    plsc.subcore_barrier()
    for k in range(NS):
        pltpu.sync_copy(shared.at[k, pl.ds(sid * SL, SL)], red.at[k])

    def body(j, _):
        sl = pl.ds(j * 16, 16)
        acc = red[0, sl]
        for k in range(1, NS):
            acc = acc + red[k, sl]
        outb[sl] = acc
        return _

    lax.fori_loop(0, SL // 16, body, None)
    pltpu.sync_copy(outb, out_hbm.at[cid, pl.ds(sid * SL, SL)])


@functools.partial(
    pl.kernel,
    out_type=jax.ShapeDtypeStruct((NC, NPAD), jnp.float32),
    mesh=_mesh,
    compiler_params=_sc_params,
    scratch_types=[
        pltpu.VMEM((NPAD,), jnp.float32),
        pltpu.VMEM((NCHT, CHE), jnp.int32),
        pltpu.VMEM_SHARED((NS, NPAD), jnp.float32),
        pltpu.VMEM((NS, SL), jnp.float32),
        pltpu.VMEM((SL,), jnp.float32),
    ],
)
def _deg_call(dstc_hbm, out_hbm, hist, dif, shared, red, outb):
    wid = _wid()
    pltpu.sync_copy(dstc_hbm.at[wid], dif)
    _zero_vmem_1d(hist, NPAD)
    ones = jnp.ones((16,), jnp.float32)

    def body(i, _):
        def inner(j, _2):
            idx = dif[i, pl.ds(j * 16, 16)]
            plsc.addupdate_scatter(hist, [idx], ones)
            return _2

        lax.fori_loop(0, CHE // 16, inner, None)
        return _

    lax.fori_loop(0, NCHT, body, None)
    _combine_and_store(hist, shared, red, outb, out_hbm)


@functools.partial(
    pl.kernel,
    out_type=jax.ShapeDtypeStruct((NC, NPAD), jnp.float32),
    mesh=_mesh,
    compiler_params=_sc_params,
    scratch_types=[
        pltpu.VMEM((NPAD,), jnp.float32),
        pltpu.VMEM((NPAD,), jnp.float32),
        pltpu.VMEM((NCHT, CHE), jnp.int32),
        pltpu.VMEM((NCHT, CHE), jnp.int32),
        pltpu.VMEM_SHARED((NS, NPAD), jnp.float32),
        pltpu.VMEM((NS, SL), jnp.float32),
        pltpu.VMEM((SL,), jnp.float32),
    ],
)
def _g_call(dinv_hbm, srcc_hbm, dstc_hbm, out_hbm, dtab, hist, sif, dif,
            shared, red, outb):
    wid = _wid()
    pltpu.sync_copy(srcc_hbm.at[wid], sif)
    pltpu.sync_copy(dstc_hbm.at[wid], dif)
    pltpu.sync_copy(dinv_hbm, dtab)
    _zero_vmem_1d(hist, NPAD)

    def body(i, _):
        def inner(j, _2):
            sl = pl.ds(j * 16, 16)
            vals = plsc.load_gather(dtab, [dif[i, sl]])
            plsc.addupdate_scatter(hist, [sif[i, sl]], vals)
            return _2

        lax.fori_loop(0, CHE // 16, inner, None)
        return _

    lax.fori_loop(0, NCHT, body, None)
    _combine_and_store(hist, shared, red, outb, out_hbm)

@functools.partial(
    pl.kernel,
    out_type=jax.ShapeDtypeStruct((NC, NPAD, D1P), jnp.float32),
    mesh=_mesh,
    compiler_params=_sc_params,
    scratch_types=[
        pltpu.VMEM((NCHT, CHE), jnp.int32),
        pltpu.VMEM((NCHT, CHE), jnp.int32),
        pltpu.VMEM((CHE, D1P), jnp.float32),
        pltpu.VMEM((CHE, D1P), jnp.float32),
        pltpu.SemaphoreType.DMA,
        pltpu.SemaphoreType.DMA,
        pltpu.VMEM_SHARED((NPAD, D1P), jnp.float32),
    ],
)
def _agg1_call(hp_hbm, srcc_hbm, dstc_hbm,
               zer_hbm, s_hbm, sidx, didx, rows0, rows1,
               sem0, sem1, acc):
    cid = lax.axis_index("c")
    sid = lax.axis_index("s")
    wid = cid * NS + sid
    pltpu.sync_copy(zer_hbm, acc.at[pl.ds(sid * SL, SL)])
    pltpu.sync_copy(srcc_hbm.at[wid], sidx)
    pltpu.sync_copy(dstc_hbm.at[wid], didx)
    plsc.subcore_barrier()

    pltpu.async_copy(hp_hbm.at[sidx.at[0]], rows0, sem0)

    def body(i, _):
        c0 = 2 * i
        c1 = c0 + 1
        pltpu.async_copy(hp_hbm.at[sidx.at[c1]], rows1, sem1)
        pltpu.make_async_copy(hp_hbm.at[sidx.at[c0]], rows0, sem0).wait()
        pltpu.sync_copy(rows0, acc.at[didx.at[c0]], add=True)

        @pl.when(i < NCHT // 2 - 1)
        def _nx():
            pltpu.async_copy(hp_hbm.at[sidx.at[c0 + 2]], rows0, sem0)

        pltpu.make_async_copy(hp_hbm.at[sidx.at[c1]], rows1, sem1).wait()
        pltpu.sync_copy(rows1, acc.at[didx.at[c1]], add=True)
        return _

    lax.fori_loop(0, NCHT // 2, body, None)
    plsc.subcore_barrier()
    pltpu.sync_copy(acc.at[pl.ds(sid * SL, SL)],
                    s_hbm.at[cid, pl.ds(sid * SL, SL)])


@functools.partial(
    pl.kernel,
    out_type=jax.ShapeDtypeStruct((NC, NPAD, D2), jnp.float32),
    mesh=_mesh,
    compiler_params=_sc_params,
    scratch_types=[
        pltpu.VMEM((NCHT, CHE), jnp.int32),
        pltpu.VMEM((NCHT, CHE), jnp.int32),
        pltpu.VMEM((CHE, D2), jnp.float32),
        pltpu.VMEM((CHE, D2), jnp.float32),
        pltpu.SemaphoreType.DMA,
        pltpu.SemaphoreType.DMA,
        pltpu.VMEM_SHARED((NPAD, D2), jnp.float32),
    ],
)
def _agg2_call(hp_hbm, srcc_hbm, dstc_hbm, zer_hbm, s_hbm, sidx, didx,
               rows0, rows1, sem0, sem1, acc):
    cid = lax.axis_index("c")
    sid = lax.axis_index("s")
    wid = cid * NS + sid
    pltpu.sync_copy(zer_hbm, acc.at[pl.ds(sid * SL, SL)])
    pltpu.sync_copy(srcc_hbm.at[wid], sidx)
    pltpu.sync_copy(dstc_hbm.at[wid], didx)
    plsc.subcore_barrier()

    pltpu.async_copy(hp_hbm.at[sidx.at[0]], rows0, sem0)

    def body(i, _):
        c0 = 2 * i
        c1 = c0 + 1
        pltpu.async_copy(hp_hbm.at[sidx.at[c1]], rows1, sem1)
        pltpu.make_async_copy(hp_hbm.at[sidx.at[c0]], rows0, sem0).wait()
        pltpu.sync_copy(rows0, acc.at[didx.at[c0]], add=True)

        @pl.when(i < NCHT // 2 - 1)
        def _nx():
            pltpu.async_copy(hp_hbm.at[sidx.at[c0 + 2]], rows0, sem0)

        pltpu.make_async_copy(hp_hbm.at[sidx.at[c1]], rows1, sem1).wait()
        pltpu.sync_copy(rows1, acc.at[didx.at[c1]], add=True)
        return _

    lax.fori_loop(0, NCHT // 2, body, None)
    plsc.subcore_barrier()
    pltpu.sync_copy(acc.at[pl.ds(sid * SL, SL)],
                    s_hbm.at[cid, pl.ds(sid * SL, SL)])


def _mm1_body(ca_ref, cb_ref, x_ref, w_ref, p_ref, dinv_ref):
    i = pl.program_id(0)
    row = lax.broadcasted_iota(jnp.int32, (BR, 1), 0) + i * BR
    deg = ca_ref[...] + cb_ref[...] + 1.0
    dv = jnp.where(row < N, lax.rsqrt(deg), 0.0)
    dinv_ref[...] = dv
    p_ref[...] = jnp.dot(x_ref[...], w_ref[...],
                         preferred_element_type=jnp.float32) * dv


def _mm2_body(sa_ref, sb_ref, p1_ref, dv_ref, b1_ref, w2_ref, out_ref):
    dv = dv_ref[...]
    h = dv * (sa_ref[...] + sb_ref[...] + p1_ref[...]) + b1_ref[...]
    h = jnp.maximum(h, 0.0)
    out_ref[...] = jnp.dot(h, w2_ref[...],
                           preferred_element_type=jnp.float32) * dv


def _fin_body(sa_ref, sb_ref, p2_ref, dv_ref, ga_ref, gb_ref, b2_ref, w3_ref,
              b3_ref, out_ref, acc_ref):
    i = pl.program_id(0)

    @pl.when(i == 0)
    def _z():
        acc_ref[...] = jnp.zeros_like(acc_ref)

    dv = dv_ref[...]
    h = jnp.maximum(dv * (sa_ref[...] + sb_ref[...] + p2_ref[...]) + b2_ref[...],
                    0.0)
    c = dv * (ga_ref[...] + gb_ref[...] + dv)
    acc_ref[...] += jnp.sum(c * h, axis=0, keepdims=True)

    @pl.when(i == GRID - 1)
    def _f():
        out_ref[...] = jnp.dot(acc_ref[...] * (1.0 / N), w3_ref[...],
                               preferred_element_type=jnp.float32) + b3_ref[...]


def _col_spec(d):
    return pl.BlockSpec((BR, d), lambda i: (i, 0))


def _const_spec(shape):
    return pl.BlockSpec(shape, lambda i: tuple(0 for _ in shape))


def kernel(x, edge_index, W1, b1, W2, b2, W3, b3):
    f32 = jnp.float32
    src = edge_index[0].astype(jnp.int32)
    dst = edge_index[1].astype(jnp.int32)
    npd = EPAD - E
    pidx = jnp.arange(npd, dtype=jnp.int32)
    srcp = jnp.concatenate([src, pidx % N])
    dstp = jnp.concatenate([dst, N + (pidx % (NPAD - N))])
    srcc = srcp.reshape(NT, NCHT, CHE)
    dstc = dstp.reshape(NT, NCHT, CHE)
    xpad = jnp.zeros((NPAD, D_IN), f32).at[:N].set(x)
    W1p = jnp.zeros((D_IN, D1P), f32).at[:, :D1].set(W1)
    b1p = jnp.zeros((1, D1P), f32).at[0, :D1].set(b1)
    W2p = jnp.zeros((D1P, D2), f32).at[:D1].set(W2)

    cnt2 = _deg_call(dstc)

    P1p, dinv = pl.pallas_call(
        _mm1_body,
        grid=(GRID,),
        in_specs=[_col_spec(1), _col_spec(1), _col_spec(D_IN),
                  _const_spec((D_IN, D1P))],
        out_specs=[_col_spec(D1P), _col_spec(1)],
        out_shape=[jax.ShapeDtypeStruct((NPAD, D1P), f32),
                   jax.ShapeDtypeStruct((NPAD, 1), f32)],
    )(cnt2[0][:, None], cnt2[1][:, None], xpad, W1p)

    z1 = jnp.zeros((SL, D1P), f32)
    g2 = _g_call(dinv[:, 0], srcc, dstc)
    S1 = _agg1_call(P1p, srcc, dstc, z1)

    P2p = pl.pallas_call(
        _mm2_body,
        grid=(GRID,),
        in_specs=[_col_spec(D1P), _col_spec(D1P), _col_spec(D1P), _col_spec(1),
                  _const_spec((1, D1P)), _const_spec((D1P, D2))],
        out_specs=_col_spec(D2),
        out_shape=jax.ShapeDtypeStruct((NPAD, D2), f32),
    )(S1[0], S1[1], P1p, dinv, b1p, W2p)

    z2 = jnp.zeros((SL, D2), f32)
    S2 = _agg2_call(P2p, srcc, dstc, z2)

    out = pl.pallas_call(
        _fin_body,
        grid=(GRID,),
        in_specs=[_col_spec(D2), _col_spec(D2), _col_spec(D2), _col_spec(1),
                  _col_spec(1), _col_spec(1), _const_spec((1, D2)),
                  _const_spec((D2, D3)), _const_spec((1, D3))],
        out_specs=_const_spec((1, D3)),
        out_shape=jax.ShapeDtypeStruct((1, D3), f32),
        scratch_shapes=[pltpu.VMEM((1, D2), f32)],
    )(S2[0], S2[1], P2p, dinv, g2[0][:, None], g2[1][:, None],
      b2[None, :], W3, b3[None, :])

    return out
